# baseline (speedup 1.0000x reference)
import functools

import jax
import jax.numpy as jnp
from jax.experimental import pallas as pl
from jax.experimental.pallas import tpu as pltpu
from jax.experimental.pallas import tpu_sc as plsc

_CH = 32
_LANES = 16


def kernel(x, inds):
    n_rows, n_cols = x.shape
    k = inds.shape[0]
    n_groups = k // _LANES
    mesh = plsc.VectorSubcoreMesh(core_axis_name="c", subcore_axis_name="s")

    @functools.partial(
        pl.kernel,
        out_type=jax.ShapeDtypeStruct((n_rows * k,), x.dtype),
        mesh=mesh,
        scratch_types=[pltpu.VMEM((k,), jnp.int32)],
        compiler_params=pltpu.CompilerParams(needs_layout_passes=False),
    )
    def sc_gather(x_hbm, inds_hbm, o_hbm, inds_v):
        pltpu.sync_copy(inds_hbm, inds_v)

        def body(in_v, out_v):
            cols = [inds_v[pl.ds(g * _LANES, _LANES)] for g in range(n_groups)]

            @pl.loop(0, _CH)
            def _(r):
                in_base = r * n_cols
                out_base = r * k
                for g in range(n_groups):
                    vals = plsc.load_gather(in_v, [in_base + cols[g]])
                    out_v[pl.ds(out_base + g * _LANES, _LANES)] = vals

        pltpu.emit_pipeline(
            body,
            grid=(n_rows // _CH,),
            in_specs=[pl.BlockSpec((_CH * n_cols,), lambda i: (i,))],
            out_specs=[pl.BlockSpec((_CH * k,), lambda i: (i,))],
            core_axis_name=("c", "s"),
            dimension_semantics=(pltpu.PARALLEL,),
        )(x_hbm, o_hbm)

    return sc_gather(x.reshape(-1), inds).reshape(n_rows, k)


# speedup vs baseline: 1.9619x; 1.9619x over previous
import functools

import jax
import jax.numpy as jnp
from jax.experimental import pallas as pl
from jax.experimental.pallas import tpu as pltpu
from jax.experimental.pallas import tpu_sc as plsc

_CH = 32
_LANES = 16


def kernel(x, inds):
    n_rows, n_cols = x.shape
    k = inds.shape[0]
    n_groups = k // _LANES
    mesh = plsc.VectorSubcoreMesh(core_axis_name="c", subcore_axis_name="s")

    @functools.partial(
        pl.kernel,
        out_type=jax.ShapeDtypeStruct((n_rows, k), x.dtype),
        mesh=mesh,
        scratch_types=[pltpu.VMEM((k,), jnp.int32)],
        compiler_params=pltpu.CompilerParams(needs_layout_passes=False),
    )
    def sc_gather(x_hbm, inds_hbm, o_hbm, inds_v):
        pltpu.sync_copy(inds_hbm, inds_v)

        def body(in_v, out_v):
            cols = [inds_v[pl.ds(g * _LANES, _LANES)] for g in range(n_groups)]

            @pl.loop(0, _CH)
            def _(r):
                row = jnp.full((_LANES,), r, jnp.int32)
                for g in range(n_groups):
                    vals = plsc.load_gather(in_v, [row, cols[g]])
                    out_v[r, pl.ds(g * _LANES, _LANES)] = vals

        pltpu.emit_pipeline(
            body,
            grid=(n_rows // _CH,),
            in_specs=[pl.BlockSpec((_CH, n_cols), lambda i: (i, 0))],
            out_specs=[pl.BlockSpec((_CH, k), lambda i: (i, 0))],
            core_axis_name=("c", "s"),
            dimension_semantics=(pltpu.PARALLEL,),
        )(x_hbm, o_hbm)

    return sc_gather(x, inds)


# speedup vs baseline: 2.1058x; 1.0733x over previous
import functools

import jax
import jax.numpy as jnp
from jax import lax
from jax.experimental import pallas as pl
from jax.experimental.pallas import tpu as pltpu
from jax.experimental.pallas import tpu_sc as plsc

_CH = 32
_LANES = 16
_TC_ROWS = 10240
_TC_BLK = 2048


def _gather_mm(x_ref, m_ref, o_ref):
    o_ref[...] = jnp.dot(x_ref[...], m_ref[...],
                         preferred_element_type=jnp.float32)


def _tc_part(x, inds):
    n_rows, n_cols = x.shape
    k = inds.shape[0]
    m = (inds[None, :] == jnp.arange(n_cols, dtype=inds.dtype)[:, None])
    m = m.astype(x.dtype)
    return pl.pallas_call(
        _gather_mm,
        grid=(_TC_ROWS // _TC_BLK,),
        in_specs=[
            pl.BlockSpec((_TC_BLK, n_cols), lambda i: (i, 0)),
            pl.BlockSpec((n_cols, k), lambda i: (0, 0)),
        ],
        out_specs=pl.BlockSpec((_TC_BLK, k), lambda i: (i, 0)),
        out_shape=jax.ShapeDtypeStruct((_TC_ROWS, k), x.dtype),
    )(x, m)


def _sc_part(x, inds):
    n_rows, n_cols = x.shape
    k = inds.shape[0]
    n_groups = k // _LANES
    blk0 = _TC_ROWS // _CH
    mesh = plsc.VectorSubcoreMesh(core_axis_name="c", subcore_axis_name="s")

    @functools.partial(
        pl.kernel,
        out_type=jax.ShapeDtypeStruct((n_rows, k), x.dtype),
        mesh=mesh,
        scratch_types=[pltpu.VMEM((k,), jnp.int32)],
        compiler_params=pltpu.CompilerParams(needs_layout_passes=False),
    )
    def sc_gather(x_hbm, inds_hbm, o_hbm, inds_v):
        pltpu.sync_copy(inds_hbm, inds_v)

        def body(in_v, out_v):
            cols = [inds_v[pl.ds(g * _LANES, _LANES)] for g in range(n_groups)]

            @pl.loop(0, _CH)
            def _(r):
                row = jnp.full((_LANES,), r, jnp.int32)
                for g in range(n_groups):
                    vals = plsc.load_gather(in_v, [row, cols[g]])
                    out_v[r, pl.ds(g * _LANES, _LANES)] = vals

        pltpu.emit_pipeline(
            body,
            grid=((n_rows - _TC_ROWS) // _CH,),
            in_specs=[pl.BlockSpec((_CH, n_cols), lambda i: (i + blk0, 0))],
            out_specs=[pl.BlockSpec((_CH, k), lambda i: (i + blk0, 0))],
            core_axis_name=("c", "s"),
            dimension_semantics=(pltpu.PARALLEL,),
        )(x_hbm, o_hbm)

    return sc_gather(x, inds)


def kernel(x, inds):
    sc_out = _sc_part(x, inds)
    tc_out = _tc_part(x, inds)
    return lax.dynamic_update_slice(sc_out, tc_out, (0, 0))


# speedup vs baseline: 3.7703x; 1.7905x over previous
import jax
import jax.numpy as jnp
from jax.experimental import pallas as pl


def _gather_mm(x_ref, m_ref, o_ref):
    o_ref[...] = jnp.dot(x_ref[...], m_ref[...],
                         preferred_element_type=jnp.float32)


def kernel(x, inds):
    n_rows, n_cols = x.shape
    k = inds.shape[0]
    m = (inds[None, :] == jnp.arange(n_cols, dtype=inds.dtype)[:, None])
    m = m.astype(x.dtype)

    block_rows = 2048
    grid = (n_rows // block_rows,)
    return pl.pallas_call(
        _gather_mm,
        grid=grid,
        in_specs=[
            pl.BlockSpec((block_rows, n_cols), lambda i: (i, 0)),
            pl.BlockSpec((n_cols, k), lambda i: (0, 0)),
        ],
        out_specs=pl.BlockSpec((block_rows, k), lambda i: (i, 0)),
        out_shape=jax.ShapeDtypeStruct((n_rows, k), x.dtype),
    )(x, m)
